# Initial kernel scaffold; baseline (speedup 1.0000x reference)
import jax
import jax.numpy as jnp
from jax.experimental import pallas as pl


def kernel(x, edge_attr, W, b, edge_index):
    raise NotImplementedError("write your pallas kernel here")


# speedup vs baseline: 2.3501x; 2.3501x over previous
import functools

import jax
import jax.numpy as jnp
from jax import lax
from jax.experimental import pallas as pl
from jax.experimental.pallas import tpu as pltpu
from jax.experimental.pallas import tpu_sc as plsc


def _proj_body(x_ref, wr_ref, ws_ref, xr_ref, xs_ref):
    xb = x_ref[...]
    xr_ref[...] = jnp.dot(xb, wr_ref[...], preferred_element_type=jnp.float32)
    xs_ref[...] = jnp.dot(xb, ws_ref[...], preferred_element_type=jnp.float32)


def _eb_body(ea_ref, we_ref, b_ref, eb_ref):
    eb_ref[...] = (
        jnp.dot(ea_ref[...], we_ref[...], preferred_element_type=jnp.float32)
        + b_ref[...]
    )


def kernel(x, edge_attr, W, b, edge_index):
    N, F = x.shape
    E, DE = edge_attr.shape
    DO = W.shape[1]

    We = W[:DE]
    Wr = W[DE : DE + F]
    Ws = W[DE + F :]
    idx = edge_index.astype(jnp.int32)
    idx_r = idx[0]
    idx_s = idx[1]
    b2 = b.reshape(1, DO)

    BN = 1000
    xr, xs = pl.pallas_call(
        _proj_body,
        grid=(N // BN,),
        in_specs=[
            pl.BlockSpec((BN, F), lambda i: (i, 0)),
            pl.BlockSpec((F, DO), lambda i: (0, 0)),
            pl.BlockSpec((F, DO), lambda i: (0, 0)),
        ],
        out_specs=[
            pl.BlockSpec((BN, DO), lambda i: (i, 0)),
            pl.BlockSpec((BN, DO), lambda i: (i, 0)),
        ],
        out_shape=[
            jax.ShapeDtypeStruct((N, DO), jnp.float32),
            jax.ShapeDtypeStruct((N, DO), jnp.float32),
        ],
    )(x, Wr, Ws)

    BE = 2000
    eb = pl.pallas_call(
        _eb_body,
        grid=(E // BE,),
        in_specs=[
            pl.BlockSpec((BE, DE), lambda i: (i, 0)),
            pl.BlockSpec((DE, DO), lambda i: (0, 0)),
            pl.BlockSpec((1, DO), lambda i: (0, 0)),
        ],
        out_specs=pl.BlockSpec((BE, DO), lambda i: (i, 0)),
        out_shape=jax.ShapeDtypeStruct((E, DO), jnp.float32),
    )(edge_attr, We, b2)

    info = plsc.get_sparse_core_info()
    NC, NS = info.num_cores, info.num_subcores
    NW = NC * NS
    per_w = E // NW
    C = 80
    n_chunks = per_w // C

    mesh = plsc.VectorSubcoreMesh(core_axis_name="c", subcore_axis_name="s")

    @functools.partial(
        pl.kernel,
        mesh=mesh,
        out_type=jax.ShapeDtypeStruct((E, DO), jnp.float32),
        scratch_types=[
            pltpu.VMEM((C,), jnp.int32),
            pltpu.VMEM((C,), jnp.int32),
            pltpu.VMEM((C, DO), jnp.float32),
            pltpu.SemaphoreType.DMA,
        ],
    )
    def _sc_combine(
        idx_r_hbm, idx_s_hbm, xr_hbm, xs_hbm, eb_hbm, out_hbm,
        idx_r_v, idx_s_v, acc_v, sem,
    ):
        wid = lax.axis_index("s") * NC + lax.axis_index("c")

        def chunk(ci, carry):
            base = wid * per_w + ci * C
            pltpu.sync_copy(idx_r_hbm.at[pl.ds(base, C)], idx_r_v)
            pltpu.sync_copy(idx_s_hbm.at[pl.ds(base, C)], idx_s_v)
            pltpu.sync_copy(eb_hbm.at[pl.ds(base, C)], acc_v)
            d1 = pltpu.async_copy(xr_hbm.at[idx_r_v], acc_v, sem, add=True)
            d2 = pltpu.async_copy(xs_hbm.at[idx_s_v], acc_v, sem, add=True)
            d1.wait()
            d2.wait()
            pltpu.sync_copy(acc_v, out_hbm.at[pl.ds(base, C)])
            return carry

        lax.fori_loop(0, n_chunks, chunk, 0)

    return _sc_combine(idx_r, idx_s, xr, xs, eb)


# speedup vs baseline: 3.2563x; 1.3856x over previous
import functools

import jax
import jax.numpy as jnp
from jax import lax
from jax.experimental import pallas as pl
from jax.experimental.pallas import tpu as pltpu
from jax.experimental.pallas import tpu_sc as plsc


def _proj_body(x_ref, wr_ref, ws_ref, xr_ref, xs_ref):
    xb = x_ref[...]
    xr_ref[...] = jnp.dot(xb, wr_ref[...], preferred_element_type=jnp.float32)
    xs_ref[...] = jnp.dot(xb, ws_ref[...], preferred_element_type=jnp.float32)


def _eb_body(ea_ref, we_ref, b_ref, eb_ref):
    eb_ref[...] = (
        jnp.dot(ea_ref[...], we_ref[...], preferred_element_type=jnp.float32)
        + b_ref[...]
    )


def kernel(x, edge_attr, W, b, edge_index):
    N, F = x.shape
    E, DE = edge_attr.shape
    DO = W.shape[1]

    We = W[:DE]
    Wr = W[DE : DE + F]
    Ws = W[DE + F :]
    idx = edge_index.astype(jnp.int32)
    idx_r = idx[0]
    idx_s = idx[1]
    b2 = b.reshape(1, DO)

    BN = 1000
    xr, xs = pl.pallas_call(
        _proj_body,
        grid=(N // BN,),
        in_specs=[
            pl.BlockSpec((BN, F), lambda i: (i, 0)),
            pl.BlockSpec((F, DO), lambda i: (0, 0)),
            pl.BlockSpec((F, DO), lambda i: (0, 0)),
        ],
        out_specs=[
            pl.BlockSpec((BN, DO), lambda i: (i, 0)),
            pl.BlockSpec((BN, DO), lambda i: (i, 0)),
        ],
        out_shape=[
            jax.ShapeDtypeStruct((N, DO), jnp.float32),
            jax.ShapeDtypeStruct((N, DO), jnp.float32),
        ],
    )(x, Wr, Ws)

    BE = 2000
    eb = pl.pallas_call(
        _eb_body,
        grid=(E // BE,),
        in_specs=[
            pl.BlockSpec((BE, DE), lambda i: (i, 0)),
            pl.BlockSpec((DE, DO), lambda i: (0, 0)),
            pl.BlockSpec((1, DO), lambda i: (0, 0)),
        ],
        out_specs=pl.BlockSpec((BE, DO), lambda i: (i, 0)),
        out_shape=jax.ShapeDtypeStruct((E, DO), jnp.float32),
    )(edge_attr, We, b2)

    info = plsc.get_sparse_core_info()
    NC, NS = info.num_cores, info.num_subcores
    NW = NC * NS
    per_w = E // NW
    C = 128
    n_full = per_w // C
    rem = per_w - n_full * C
    n_pairs = n_full // 2
    odd_tail = n_full - 2 * n_pairs

    mesh = plsc.VectorSubcoreMesh(core_axis_name="c", subcore_axis_name="s")

    @functools.partial(
        pl.kernel,
        mesh=mesh,
        out_type=jax.ShapeDtypeStruct((E, DO), jnp.float32),
        scratch_types=[
            pltpu.VMEM((C,), jnp.int32),
            pltpu.VMEM((C,), jnp.int32),
            pltpu.VMEM((C, DO), jnp.float32),
            pltpu.VMEM((C,), jnp.int32),
            pltpu.VMEM((C,), jnp.int32),
            pltpu.VMEM((C, DO), jnp.float32),
            pltpu.VMEM((max(rem, 8),), jnp.int32),
            pltpu.VMEM((max(rem, 8),), jnp.int32),
            pltpu.VMEM((max(rem, 8), DO), jnp.float32),
            pltpu.SemaphoreType.DMA,
            pltpu.SemaphoreType.DMA,
            pltpu.SemaphoreType.DMA,
            pltpu.SemaphoreType.DMA,
            pltpu.SemaphoreType.DMA,
        ],
    )
    def _sc_combine(
        idx_r_hbm, idx_s_hbm, xr_hbm, xs_hbm, eb_hbm, out_hbm,
        idxr0, idxs0, acc0, idxr1, idxs1, acc1, idxr_t, idxs_t, acc_t,
        lsem0, lsem1, osem0, osem1, gsem,
    ):
        wid = lax.axis_index("s") * NC + lax.axis_index("c")
        w_base = wid * per_w
        idxr = (idxr0, idxr1)
        idxs = (idxs0, idxs1)
        acc = (acc0, acc1)
        lsem = (lsem0, lsem1)
        osem = (osem0, osem1)

        def loads(ci, slot):
            base = w_base + ci * C
            return (
                pltpu.make_async_copy(
                    idx_r_hbm.at[pl.ds(base, C)], idxr[slot], lsem[slot]),
                pltpu.make_async_copy(
                    idx_s_hbm.at[pl.ds(base, C)], idxs[slot], lsem[slot]),
                pltpu.make_async_copy(
                    eb_hbm.at[pl.ds(base, C)], acc[slot], lsem[slot]),
            )

        def store(ci, slot):
            base = w_base + ci * C
            return pltpu.make_async_copy(
                acc[slot], out_hbm.at[pl.ds(base, C)], osem[slot])

        def issue(descs):
            for d in descs:
                d.start()

        def chunk_step(ci, slot):
            @pl.when(ci >= 1)
            def _():
                store(ci - 1, 1 - slot).wait()

            @pl.when(ci + 1 < n_full)
            def _():
                issue(loads(ci + 1, 1 - slot))

            d1 = pltpu.async_copy(
                xr_hbm.at[idxr[slot]], acc[slot], gsem, add=True)
            d2 = pltpu.async_copy(
                xs_hbm.at[idxs[slot]], acc[slot], gsem, add=True)
            d1.wait()
            d2.wait()
            store(ci, slot).start()

        issue(loads(0, 0))

        def pair_body(p, carry):
            for b in range(2):
                ci = 2 * p + b
                for d in loads(ci, b):
                    d.wait()
                chunk_step(ci, b)
            return carry

        lax.fori_loop(0, n_pairs, pair_body, 0)

        if odd_tail:
            ci = 2 * n_pairs
            for d in loads(ci, 0):
                d.wait()
            chunk_step(ci, 0)

        last = n_full - 1
        store(last, last % 2).wait()

        if rem:
            base = w_base + n_full * C
            pltpu.sync_copy(idx_r_hbm.at[pl.ds(base, rem)], idxr_t)
            pltpu.sync_copy(idx_s_hbm.at[pl.ds(base, rem)], idxs_t)
            pltpu.sync_copy(eb_hbm.at[pl.ds(base, rem)], acc_t)
            d1 = pltpu.async_copy(xr_hbm.at[idxr_t], acc_t, gsem, add=True)
            d2 = pltpu.async_copy(xs_hbm.at[idxs_t], acc_t, gsem, add=True)
            d1.wait()
            d2.wait()
            pltpu.sync_copy(acc_t, out_hbm.at[pl.ds(base, rem)])

    return _sc_combine(idx_r, idx_s, xr, xs, eb)


# speedup vs baseline: 3.6998x; 1.1362x over previous
import functools

import jax
import jax.numpy as jnp
from jax import lax
from jax.experimental import pallas as pl
from jax.experimental.pallas import tpu as pltpu
from jax.experimental.pallas import tpu_sc as plsc


def _proj_body(x_ref, wr_ref, ws_ref, xr_ref, xs_ref):
    xb = x_ref[...]
    xr_ref[...] = jnp.dot(xb, wr_ref[...], preferred_element_type=jnp.float32)
    xs_ref[...] = jnp.dot(xb, ws_ref[...], preferred_element_type=jnp.float32)


def _eb_body(ea_ref, we_ref, b_ref, eb_ref):
    eb_ref[...] = (
        jnp.dot(ea_ref[...], we_ref[...], preferred_element_type=jnp.float32)
        + b_ref[...]
    )


def kernel(x, edge_attr, W, b, edge_index):
    N, F = x.shape
    E, DE = edge_attr.shape
    DO = W.shape[1]

    We = W[:DE]
    Wr = W[DE : DE + F]
    Ws = W[DE + F :]
    idx = edge_index.astype(jnp.int32)
    idx_r = idx[0]
    idx_s = idx[1]
    b2 = b.reshape(1, DO)

    BN = 1000
    xr, xs = pl.pallas_call(
        _proj_body,
        grid=(N // BN,),
        in_specs=[
            pl.BlockSpec((BN, F), lambda i: (i, 0)),
            pl.BlockSpec((F, DO), lambda i: (0, 0)),
            pl.BlockSpec((F, DO), lambda i: (0, 0)),
        ],
        out_specs=[
            pl.BlockSpec((BN, DO), lambda i: (i, 0)),
            pl.BlockSpec((BN, DO), lambda i: (i, 0)),
        ],
        out_shape=[
            jax.ShapeDtypeStruct((N, DO), jnp.float32),
            jax.ShapeDtypeStruct((N, DO), jnp.float32),
        ],
    )(x, Wr, Ws)

    BE = 16000
    eb = pl.pallas_call(
        _eb_body,
        grid=(E // BE,),
        in_specs=[
            pl.BlockSpec((BE, DE), lambda i: (i, 0)),
            pl.BlockSpec((DE, DO), lambda i: (0, 0)),
            pl.BlockSpec((1, DO), lambda i: (0, 0)),
        ],
        out_specs=pl.BlockSpec((BE, DO), lambda i: (i, 0)),
        out_shape=jax.ShapeDtypeStruct((E, DO), jnp.float32),
    )(edge_attr, We, b2)

    info = plsc.get_sparse_core_info()
    NC, NS = info.num_cores, info.num_subcores
    NW = NC * NS
    per_w = E // NW
    C = 128
    n_full = per_w // C
    rem = per_w - n_full * C
    n_pairs = n_full // 2
    odd_tail = n_full - 2 * n_pairs

    mesh = plsc.VectorSubcoreMesh(core_axis_name="c", subcore_axis_name="s")

    @functools.partial(
        pl.kernel,
        mesh=mesh,
        out_type=jax.ShapeDtypeStruct((E, DO), jnp.float32),
        scratch_types=[
            pltpu.VMEM((C,), jnp.int32),
            pltpu.VMEM((C,), jnp.int32),
            pltpu.VMEM((C, DO), jnp.float32),
            pltpu.VMEM((C,), jnp.int32),
            pltpu.VMEM((C,), jnp.int32),
            pltpu.VMEM((C, DO), jnp.float32),
            pltpu.VMEM((max(rem, 8),), jnp.int32),
            pltpu.VMEM((max(rem, 8),), jnp.int32),
            pltpu.VMEM((max(rem, 8), DO), jnp.float32),
            pltpu.SemaphoreType.DMA,
            pltpu.SemaphoreType.DMA,
            pltpu.SemaphoreType.DMA,
            pltpu.SemaphoreType.DMA,
            pltpu.SemaphoreType.DMA,
        ],
    )
    def _sc_combine(
        idx_r_hbm, idx_s_hbm, xr_hbm, xs_hbm, eb_hbm, out_hbm,
        idxr0, idxs0, acc0, idxr1, idxs1, acc1, idxr_t, idxs_t, acc_t,
        lsem0, lsem1, osem0, osem1, gsem,
    ):
        wid = lax.axis_index("s") * NC + lax.axis_index("c")
        w_base = wid * per_w
        idxr = (idxr0, idxr1)
        idxs = (idxs0, idxs1)
        acc = (acc0, acc1)
        lsem = (lsem0, lsem1)
        osem = (osem0, osem1)

        def loads(ci, slot):
            base = w_base + ci * C
            return (
                pltpu.make_async_copy(
                    idx_r_hbm.at[pl.ds(base, C)], idxr[slot], lsem[slot]),
                pltpu.make_async_copy(
                    idx_s_hbm.at[pl.ds(base, C)], idxs[slot], lsem[slot]),
                pltpu.make_async_copy(
                    eb_hbm.at[pl.ds(base, C)], acc[slot], lsem[slot]),
            )

        def store(ci, slot):
            base = w_base + ci * C
            return pltpu.make_async_copy(
                acc[slot], out_hbm.at[pl.ds(base, C)], osem[slot])

        def issue(descs):
            for d in descs:
                d.start()

        def chunk_step(ci, slot):
            @pl.when(ci >= 1)
            def _():
                store(ci - 1, 1 - slot).wait()

            @pl.when(ci + 1 < n_full)
            def _():
                issue(loads(ci + 1, 1 - slot))

            d1 = pltpu.async_copy(
                xr_hbm.at[idxr[slot]], acc[slot], gsem, add=True)
            d2 = pltpu.async_copy(
                xs_hbm.at[idxs[slot]], acc[slot], gsem, add=True)
            d1.wait()
            d2.wait()
            store(ci, slot).start()

        issue(loads(0, 0))

        def pair_body(p, carry):
            for b in range(2):
                ci = 2 * p + b
                for d in loads(ci, b):
                    d.wait()
                chunk_step(ci, b)
            return carry

        lax.fori_loop(0, n_pairs, pair_body, 0)

        if odd_tail:
            ci = 2 * n_pairs
            for d in loads(ci, 0):
                d.wait()
            chunk_step(ci, 0)

        last = n_full - 1
        store(last, last % 2).wait()

        if rem:
            base = w_base + n_full * C
            pltpu.sync_copy(idx_r_hbm.at[pl.ds(base, rem)], idxr_t)
            pltpu.sync_copy(idx_s_hbm.at[pl.ds(base, rem)], idxs_t)
            pltpu.sync_copy(eb_hbm.at[pl.ds(base, rem)], acc_t)
            d1 = pltpu.async_copy(xr_hbm.at[idxr_t], acc_t, gsem, add=True)
            d2 = pltpu.async_copy(xs_hbm.at[idxs_t], acc_t, gsem, add=True)
            d1.wait()
            d2.wait()
            pltpu.sync_copy(acc_t, out_hbm.at[pl.ds(base, rem)])

    return _sc_combine(idx_r, idx_s, xr, xs, eb)
